# baseline (speedup 1.0000x reference)
import functools

import jax
import jax.numpy as jnp
from jax import lax
from jax.experimental import pallas as pl
from jax.experimental.pallas import tpu as pltpu
from jax.experimental.pallas import tpu_sc as plsc

N_NODES = 10000
D_IN = 128
HID = 256
NC = 64

NUM_SC = 2
NUM_TILES = 16
LANES = 16

NPAD = 10240
ROWS_PER_TILE = NPAD // NUM_TILES

EA = N_NODES + 320000
EB = 128
EA_PAD = 344064
NB = EA_PAD // (NUM_TILES * EB)
ES = NB * EB

EPS = 1e-16


def _mm1_body(x_ref, w_ref, av_ref, hl_ref, hr_ref, ab_ref):
    h = jnp.dot(x_ref[...], w_ref[...], preferred_element_type=jnp.float32)
    hl_ref[...] = h[:, : HID // 2]
    hr_ref[...] = h[:, HID // 2 :]
    ab_ref[...] = lax.dot_general(av_ref[...], h, (((1,), (1,)), ((), ())))


def _mm2_body(il_ref, ir_ref, wa_ref, wb_ref, av_ref, h2_ref, ab_ref):
    h = jnp.dot(il_ref[...], wa_ref[...], preferred_element_type=jnp.float32)
    h = h + jnp.dot(ir_ref[...], wb_ref[...], preferred_element_type=jnp.float32)
    h2_ref[...] = h
    ab_ref[...] = lax.dot_general(av_ref[...], h, (((1,), (1,)), ((), ())))


_BN = 1024


def _mm1(x, w, av):
    grid = (NPAD // _BN,)
    return pl.pallas_call(
        _mm1_body,
        grid=grid,
        in_specs=[
            pl.BlockSpec((_BN, D_IN), lambda i: (i, 0)),
            pl.BlockSpec((D_IN, HID), lambda i: (0, 0)),
            pl.BlockSpec((2, HID), lambda i: (0, 0)),
        ],
        out_specs=[
            pl.BlockSpec((_BN, HID // 2), lambda i: (i, 0)),
            pl.BlockSpec((_BN, HID // 2), lambda i: (i, 0)),
            pl.BlockSpec((2, _BN), lambda i: (0, i)),
        ],
        out_shape=[
            jax.ShapeDtypeStruct((NPAD, HID // 2), jnp.float32),
            jax.ShapeDtypeStruct((NPAD, HID // 2), jnp.float32),
            jax.ShapeDtypeStruct((2, NPAD), jnp.float32),
        ],
    )(x, w, av)


def _mm2(il, ir, wa, wb, av):
    grid = (NPAD // _BN,)
    return pl.pallas_call(
        _mm2_body,
        grid=grid,
        in_specs=[
            pl.BlockSpec((_BN, HID // 2), lambda i: (i, 0)),
            pl.BlockSpec((_BN, HID // 2), lambda i: (i, 0)),
            pl.BlockSpec((HID // 2, NC), lambda i: (0, 0)),
            pl.BlockSpec((HID // 2, NC), lambda i: (0, 0)),
            pl.BlockSpec((2, NC), lambda i: (0, 0)),
        ],
        out_specs=[
            pl.BlockSpec((_BN, NC), lambda i: (i, 0)),
            pl.BlockSpec((2, _BN), lambda i: (0, i)),
        ],
        out_shape=[
            jax.ShapeDtypeStruct((NPAD, NC), jnp.float32),
            jax.ShapeDtypeStruct((2, NPAD), jnp.float32),
        ],
    )(il, ir, wa, wb, av)


def _lane_bcast(vec, l):
    idx = jnp.full((16, 1), l, jnp.int32)
    return lax.gather(
        vec, idx,
        lax.GatherDimensionNumbers(offset_dims=(), collapsed_slice_dims=(0,),
                                   start_index_map=(0,)),
        (1,), mode=lax.GatherScatterMode.PROMISE_IN_BOUNDS)


def _make_sc_gat(ch, relu_out, eb, cb):
    NB2 = ES // eb
    NCHUNK = NB2 // cb
    mesh = plsc.VectorSubcoreMesh(
        core_axis_name="c", subcore_axis_name="s",
        num_cores=NUM_SC, num_subcores=NUM_TILES)

    def body(src3, dst3, asrc, adst, hcat, bias2, outcat,
             vm_sidx, vm_didx, vm_p, vm_as, vm_ad, vm_rows, vm_bias,
             vm_den, sh_out, sh_den,
             sem_g0, sem_g1, sem_d0, sem_d1, sem_s0, sem_s1):
        c = lax.axis_index("c")
        s = lax.axis_index("s")
        row0 = s * ROWS_PER_TILE
        hbase = c * NPAD
        sem_g = [sem_g0, sem_g1]
        sem_d = [sem_d0, sem_d1]
        sem_s = [sem_s0, sem_s1]
        zc = 64
        zero16 = jnp.zeros((16,), jnp.float32)

        def load_chunk(q, slot):
            pltpu.sync_copy(src3.at[s, pl.ds(q * cb, cb)],
                            vm_sidx.at[pl.ds(slot * cb, cb)])
            pltpu.sync_copy(dst3.at[s, pl.ds(q * cb, cb)],
                            vm_didx.at[pl.ds(slot * cb, cb)])

            def brow(b, carry):
                for v in range(eb // 16):
                    vm_sidx[slot * cb + b, pl.ds(v * 16, 16)] = (
                        vm_sidx[slot * cb + b, pl.ds(v * 16, 16)] + hbase)
                return carry
            lax.fori_loop(0, cb, brow, 0)

        pltpu.sync_copy(bias2.at[c], vm_bias)
        pltpu.sync_copy(asrc, vm_as)
        pltpu.sync_copy(adst, vm_ad)
        load_chunk(0, 0)

        def zrow(j, carry):
            for k in range(ch // 16):
                vm_rows[0, j, pl.ds(k * 16, 16)] = zero16
                vm_rows[1, j, pl.ds(k * 16, 16)] = zero16
            return carry
        lax.fori_loop(0, eb, zrow, 0)
        for par in (0, 1):
            for v in range(eb // 16):
                vm_p[par, pl.ds(v * 16, 16)] = zero16

        def zden(j, carry):
            vm_den[pl.ds(j * 16, 16)] = zero16
            return carry
        lax.fori_loop(0, ROWS_PER_TILE // 16, zden, 0)

        def zout(t, carry):
            pltpu.sync_copy(vm_rows.at[0, pl.ds(0, zc)],
                            sh_out.at[pl.ds(row0 + t * zc, zc)])
            return carry
        lax.fori_loop(0, ROWS_PER_TILE // zc, zout, 0)
        pltpu.sync_copy(vm_den, sh_den.at[pl.ds(row0, ROWS_PER_TILE)])

        plsc.subcore_barrier()

        pltpu.async_copy(vm_rows.at[1], sh_out.at[vm_didx.at[0]],
                         sem_s1, add=True)
        pltpu.async_copy(vm_p.at[0], sh_den.at[vm_didx.at[0]],
                         sem_d0, add=True)
        pltpu.async_copy(vm_p.at[1], sh_den.at[vm_didx.at[0]],
                         sem_d1, add=True)
        pltpu.async_copy(hcat.at[vm_sidx.at[0]], vm_rows.at[0], sem_g0)

        def chunk_body(q, carry):
            qpar = lax.rem(q, 2)
            base = qpar * cb
            nbase = (1 - qpar) * cb
            for b in range(cb):
                par = b & 1
                nxt = 1 - par
                gb = q * cb + b
                pltpu.make_async_copy(
                    vm_p.at[par], sh_den.at[vm_didx.at[base + b]],
                    sem_d[par]).wait()
                eb0 = (s * NB2 + gb) * eb
                for v in range(eb // 16):
                    sidx = (vm_sidx[base + b, pl.ds(v * 16, 16)] - hbase)
                    didx = vm_didx[base + b, pl.ds(v * 16, 16)]
                    e = (plsc.load_gather(vm_as, [sidx])
                         + plsc.load_gather(vm_ad, [didx]))
                    e = jnp.maximum(e, 0.2 * e)
                    pos = eb0 + v * 16 + lax.iota(jnp.int32, 16)
                    p = jnp.where(pos < EA, jnp.exp(e), 0.0)
                    vm_p[par, pl.ds(v * 16, 16)] = p
                pltpu.async_copy(vm_p.at[par],
                                 sh_den.at[vm_didx.at[base + b]],
                                 sem_d[par], add=True)
                if b == cb - 1:
                    load_chunk(jnp.minimum(q + 1, NCHUNK - 1), 1 - qpar)
                pltpu.make_async_copy(
                    vm_rows.at[nxt], sh_out.at[vm_didx.at[base + b]],
                    sem_s[nxt]).wait()
                if b < cb - 1:
                    nb = base + b + 1
                else:
                    nb = nbase
                pltpu.async_copy(hcat.at[vm_sidx.at[nb]],
                                 vm_rows.at[nxt], sem_g[nxt])
                pltpu.make_async_copy(
                    hcat.at[vm_sidx.at[base + b]], vm_rows.at[par],
                    sem_g[par]).wait()

                npar = 2 if ch >= 128 else eb // 16

                def scale(g, carry3):
                    @plsc.parallel_loop(0, npar, 1)
                    def _inner(pj):
                        j2 = g * npar + pj
                        pvec = vm_p[par, pl.ds(j2 * 16, 16)]
                        for jj in range(16):
                            j = j2 * 16 + jj
                            pv = _lane_bcast(pvec, jj)
                            for k in range(ch // 16):
                                vm_rows[par, j, pl.ds(k * 16, 16)] = (
                                    vm_rows[par, j, pl.ds(k * 16, 16)] * pv)
                    return carry3
                lax.fori_loop(0, (eb // 16) // npar, scale, 0)
                pltpu.async_copy(vm_rows.at[par],
                                 sh_out.at[vm_didx.at[base + b]],
                                 sem_s[par], add=True)
            return carry
        lax.fori_loop(0, NCHUNK, chunk_body, 0)

        pltpu.make_async_copy(hcat.at[vm_sidx.at[0]], vm_rows.at[0],
                              sem_g0).wait()
        pltpu.make_async_copy(vm_rows.at[1], sh_out.at[vm_didx.at[0]],
                              sem_s1).wait()
        pltpu.make_async_copy(vm_p.at[0], sh_den.at[vm_didx.at[0]],
                              sem_d0).wait()
        pltpu.make_async_copy(vm_p.at[1], sh_den.at[vm_didx.at[0]],
                              sem_d1).wait()

        plsc.subcore_barrier()

        pltpu.sync_copy(sh_den.at[pl.ds(row0, ROWS_PER_TILE)], vm_den)

        def epi(t, carry):
            r0 = row0 + t * zc
            pltpu.sync_copy(sh_out.at[pl.ds(r0, zc)],
                            vm_rows.at[0, pl.ds(0, zc)])

            def erow(j, carry2):
                dv = plsc.load_gather(
                    vm_den, [jnp.full((16,), t * zc + j, jnp.int32)])
                rcp = 1.0 / (dv + EPS)
                for k in range(ch // 16):
                    val = (vm_rows[0, j, pl.ds(k * 16, 16)] * rcp
                           + vm_bias[pl.ds(k * 16, 16)])
                    if relu_out:
                        val = jnp.maximum(val, 0.0)
                    vm_rows[0, j, pl.ds(k * 16, 16)] = val
                return carry2
            lax.fori_loop(0, zc, erow, 0)
            pltpu.sync_copy(vm_rows.at[0, pl.ds(0, zc)],
                            outcat.at[pl.ds(hbase + r0, zc)])
            return carry
        lax.fori_loop(0, ROWS_PER_TILE // zc, epi, 0)

    return pl.kernel(
        body,
        out_type=jax.ShapeDtypeStruct((2 * NPAD, ch), jnp.float32),
        mesh=mesh,
        compiler_params=pltpu.CompilerParams(
            needs_layout_passes=False, use_tc_tiling_on_sc=False),
        scratch_types=[
            pltpu.VMEM((2 * cb, eb), jnp.int32),
            pltpu.VMEM((2 * cb, eb), jnp.int32),
            pltpu.VMEM((2, eb), jnp.float32),
            pltpu.VMEM((NPAD,), jnp.float32),
            pltpu.VMEM((NPAD,), jnp.float32),
            pltpu.VMEM((2, eb, ch), jnp.float32),
            pltpu.VMEM((ch,), jnp.float32),
            pltpu.VMEM((ROWS_PER_TILE,), jnp.float32),
            pltpu.VMEM_SHARED((NPAD, ch), jnp.float32),
            pltpu.VMEM_SHARED((NPAD,), jnp.float32),
            pltpu.SemaphoreType.DMA,
            pltpu.SemaphoreType.DMA,
            pltpu.SemaphoreType.DMA,
            pltpu.SemaphoreType.DMA,
            pltpu.SemaphoreType.DMA,
            pltpu.SemaphoreType.DMA,
        ],
    )


_sc_gat_l1 = _make_sc_gat(HID // 2, True, 64, 16)
_sc_gat_l2 = _make_sc_gat(NC // 2, False, 128, 8)


def kernel(x, edge_index, W1, a_src1, a_dst1, b1, W2, a_src2, a_dst2, b2):
    loops = jnp.arange(N_NODES, dtype=edge_index.dtype)
    ei = jnp.concatenate([edge_index, jnp.stack([loops, loops])], axis=1)
    fill = jnp.arange(EA_PAD - EA, dtype=jnp.int32) % N_NODES
    src = jnp.concatenate([ei[0], fill])
    dst = jnp.concatenate([ei[1], fill])
    src3a = src.reshape(NUM_TILES, ES // 64, 64)
    dst3a = dst.reshape(NUM_TILES, ES // 64, 64)
    src3b = src.reshape(NUM_TILES, ES // 128, 128)
    dst3b = dst.reshape(NUM_TILES, ES // 128, 128)

    x_pad = jnp.pad(x, ((0, NPAD - N_NODES), (0, 0)))

    av1 = jnp.stack([a_src1.reshape(HID), a_dst1.reshape(HID)])
    hl, hr, ab1 = _mm1(x_pad, W1, av1)
    hcat1 = jnp.concatenate([hl, hr], axis=0)
    in_cat = _sc_gat_l1(src3a, dst3a, ab1[0], ab1[1], hcat1,
                        b1.reshape(2, HID // 2))
    inl, inr = in_cat[:NPAD], in_cat[NPAD:]

    av2 = jnp.stack([a_src2.reshape(NC), a_dst2.reshape(NC)])
    h2, ab2 = _mm2(inl, inr, W2[: HID // 2], W2[HID // 2 :], av2)
    hcat2 = jnp.concatenate([h2[:, : NC // 2], h2[:, NC // 2 :]], axis=0)
    outcat = _sc_gat_l2(src3b, dst3b, ab2[0], ab2[1], hcat2,
                        b2.reshape(2, NC // 2))
    out = jnp.concatenate(
        [outcat[:N_NODES], outcat[NPAD : NPAD + N_NODES]], axis=1)
    return out


# speedup vs baseline: 1.0136x; 1.0136x over previous
import functools

import jax
import jax.numpy as jnp
from jax import lax
from jax.experimental import pallas as pl
from jax.experimental.pallas import tpu as pltpu
from jax.experimental.pallas import tpu_sc as plsc

N_NODES = 10000
D_IN = 128
HID = 256
NC = 64

NUM_SC = 2
NUM_TILES = 16
LANES = 16

NPAD = 10240
ROWS_PER_TILE = NPAD // NUM_TILES

EA = N_NODES + 320000
EB = 128
EA_PAD = 344064
NB = EA_PAD // (NUM_TILES * EB)
ES = NB * EB

EPS = 1e-16


def _mm1_body(x_ref, w_ref, av_ref, hl_ref, hr_ref, ab_ref):
    h = jnp.dot(x_ref[...], w_ref[...], preferred_element_type=jnp.float32)
    hl_ref[...] = h[:, : HID // 2]
    hr_ref[...] = h[:, HID // 2 :]
    ab_ref[...] = lax.dot_general(av_ref[...], h, (((1,), (1,)), ((), ())))


def _mm2_body(il_ref, ir_ref, wa_ref, wb_ref, av_ref, h2_ref, ab_ref):
    h = jnp.dot(il_ref[...], wa_ref[...], preferred_element_type=jnp.float32)
    h = h + jnp.dot(ir_ref[...], wb_ref[...], preferred_element_type=jnp.float32)
    h2_ref[...] = h
    ab_ref[...] = lax.dot_general(av_ref[...], h, (((1,), (1,)), ((), ())))


_BN = 1024


def _mm1(x, w, av):
    grid = (NPAD // _BN,)
    return pl.pallas_call(
        _mm1_body,
        grid=grid,
        in_specs=[
            pl.BlockSpec((_BN, D_IN), lambda i: (i, 0)),
            pl.BlockSpec((D_IN, HID), lambda i: (0, 0)),
            pl.BlockSpec((2, HID), lambda i: (0, 0)),
        ],
        out_specs=[
            pl.BlockSpec((_BN, HID // 2), lambda i: (i, 0)),
            pl.BlockSpec((_BN, HID // 2), lambda i: (i, 0)),
            pl.BlockSpec((2, _BN), lambda i: (0, i)),
        ],
        out_shape=[
            jax.ShapeDtypeStruct((NPAD, HID // 2), jnp.float32),
            jax.ShapeDtypeStruct((NPAD, HID // 2), jnp.float32),
            jax.ShapeDtypeStruct((2, NPAD), jnp.float32),
        ],
    )(x, w, av)


def _mm2(il, ir, wa, wb, av):
    grid = (NPAD // _BN,)
    return pl.pallas_call(
        _mm2_body,
        grid=grid,
        in_specs=[
            pl.BlockSpec((_BN, HID // 2), lambda i: (i, 0)),
            pl.BlockSpec((_BN, HID // 2), lambda i: (i, 0)),
            pl.BlockSpec((HID // 2, NC), lambda i: (0, 0)),
            pl.BlockSpec((HID // 2, NC), lambda i: (0, 0)),
            pl.BlockSpec((2, NC), lambda i: (0, 0)),
        ],
        out_specs=[
            pl.BlockSpec((_BN, NC), lambda i: (i, 0)),
            pl.BlockSpec((2, _BN), lambda i: (0, i)),
        ],
        out_shape=[
            jax.ShapeDtypeStruct((NPAD, NC), jnp.float32),
            jax.ShapeDtypeStruct((2, NPAD), jnp.float32),
        ],
    )(il, ir, wa, wb, av)


def _lane_bcast(vec, l):
    idx = jnp.full((16, 1), l, jnp.int32)
    return lax.gather(
        vec, idx,
        lax.GatherDimensionNumbers(offset_dims=(), collapsed_slice_dims=(0,),
                                   start_index_map=(0,)),
        (1,), mode=lax.GatherScatterMode.PROMISE_IN_BOUNDS)


def _make_sc_gat(ch, relu_out, eb, cb):
    NB2 = ES // eb
    NCHUNK = NB2 // cb
    mesh = plsc.VectorSubcoreMesh(
        core_axis_name="c", subcore_axis_name="s",
        num_cores=NUM_SC, num_subcores=NUM_TILES)

    def body(src3, dst3, asrc, adst, hcat, bias2, outcat,
             vm_sidx, vm_didx, vm_p, vm_as, vm_ad, vm_rows, vm_bias,
             vm_den, sh_out, sh_den,
             sem_g0, sem_g1, sem_d0, sem_d1, sem_s0, sem_s1):
        c = lax.axis_index("c")
        s = lax.axis_index("s")
        row0 = s * ROWS_PER_TILE
        hbase = c * NPAD
        sem_g = [sem_g0, sem_g1]
        sem_d = [sem_d0, sem_d1]
        sem_s = [sem_s0, sem_s1]
        zc = 64
        zero16 = jnp.zeros((16,), jnp.float32)

        def load_chunk(q, slot):
            pltpu.sync_copy(src3.at[s, pl.ds(q * cb, cb)],
                            vm_sidx.at[pl.ds(slot * cb, cb)])
            pltpu.sync_copy(dst3.at[s, pl.ds(q * cb, cb)],
                            vm_didx.at[pl.ds(slot * cb, cb)])

            def brow(b, carry):
                for v in range(eb // 16):
                    vm_sidx[slot * cb + b, pl.ds(v * 16, 16)] = (
                        vm_sidx[slot * cb + b, pl.ds(v * 16, 16)] + hbase)
                return carry
            lax.fori_loop(0, cb, brow, 0)

        pltpu.sync_copy(bias2.at[c], vm_bias)
        pltpu.sync_copy(asrc, vm_as)
        pltpu.sync_copy(adst, vm_ad)
        load_chunk(0, 0)

        def zrow(j, carry):
            for k in range(ch // 16):
                vm_rows[0, j, pl.ds(k * 16, 16)] = zero16
                vm_rows[1, j, pl.ds(k * 16, 16)] = zero16
            return carry
        lax.fori_loop(0, eb, zrow, 0)
        for par in (0, 1):
            for v in range(eb // 16):
                vm_p[par, pl.ds(v * 16, 16)] = zero16

        def zden(j, carry):
            vm_den[pl.ds(j * 16, 16)] = zero16
            return carry
        lax.fori_loop(0, ROWS_PER_TILE // 16, zden, 0)

        def zout(t, carry):
            pltpu.sync_copy(vm_rows.at[0, pl.ds(0, zc)],
                            sh_out.at[pl.ds(row0 + t * zc, zc)])
            return carry
        lax.fori_loop(0, ROWS_PER_TILE // zc, zout, 0)
        pltpu.sync_copy(vm_den, sh_den.at[pl.ds(row0, ROWS_PER_TILE)])

        plsc.subcore_barrier()

        pltpu.async_copy(vm_rows.at[1], sh_out.at[vm_didx.at[0]],
                         sem_s1, add=True)
        pltpu.async_copy(vm_p.at[0], sh_den.at[vm_didx.at[0]],
                         sem_d0, add=True)
        pltpu.async_copy(vm_p.at[1], sh_den.at[vm_didx.at[0]],
                         sem_d1, add=True)
        pltpu.async_copy(hcat.at[vm_sidx.at[0]], vm_rows.at[0], sem_g0)

        def chunk_body(q, carry):
            qpar = lax.rem(q, 2)
            base = qpar * cb
            nbase = (1 - qpar) * cb
            for b in range(cb):
                par = b & 1
                nxt = 1 - par
                gb = q * cb + b
                pltpu.make_async_copy(
                    vm_p.at[par], sh_den.at[vm_didx.at[base + b]],
                    sem_d[par]).wait()
                eb0 = (s * NB2 + gb) * eb
                for v in range(eb // 16):
                    sidx = (vm_sidx[base + b, pl.ds(v * 16, 16)] - hbase)
                    didx = vm_didx[base + b, pl.ds(v * 16, 16)]
                    e = (plsc.load_gather(vm_as, [sidx])
                         + plsc.load_gather(vm_ad, [didx]))
                    e = jnp.maximum(e, 0.2 * e)
                    pos = eb0 + v * 16 + lax.iota(jnp.int32, 16)
                    p = jnp.where(pos < EA, jnp.exp(e), 0.0)
                    vm_p[par, pl.ds(v * 16, 16)] = p
                pltpu.async_copy(vm_p.at[par],
                                 sh_den.at[vm_didx.at[base + b]],
                                 sem_d[par], add=True)
                if b == cb - 1:
                    load_chunk(jnp.minimum(q + 1, NCHUNK - 1), 1 - qpar)
                pltpu.make_async_copy(
                    vm_rows.at[nxt], sh_out.at[vm_didx.at[base + b]],
                    sem_s[nxt]).wait()
                if b < cb - 1:
                    nb = base + b + 1
                else:
                    nb = nbase
                pltpu.async_copy(hcat.at[vm_sidx.at[nb]],
                                 vm_rows.at[nxt], sem_g[nxt])
                pltpu.make_async_copy(
                    hcat.at[vm_sidx.at[base + b]], vm_rows.at[par],
                    sem_g[par]).wait()

                def scale(j2, carry3):
                    pvec = vm_p[par, pl.ds(j2 * 16, 16)]
                    for jj in range(16):
                        j = j2 * 16 + jj
                        pv = _lane_bcast(pvec, jj)
                        for k in range(ch // 16):
                            vm_rows[par, j, pl.ds(k * 16, 16)] = (
                                vm_rows[par, j, pl.ds(k * 16, 16)] * pv)
                    return carry3
                lax.fori_loop(0, eb // 16, scale, 0)
                pltpu.async_copy(vm_rows.at[par],
                                 sh_out.at[vm_didx.at[base + b]],
                                 sem_s[par], add=True)
            return carry
        lax.fori_loop(0, NCHUNK, chunk_body, 0)

        pltpu.make_async_copy(hcat.at[vm_sidx.at[0]], vm_rows.at[0],
                              sem_g0).wait()
        pltpu.make_async_copy(vm_rows.at[1], sh_out.at[vm_didx.at[0]],
                              sem_s1).wait()
        pltpu.make_async_copy(vm_p.at[0], sh_den.at[vm_didx.at[0]],
                              sem_d0).wait()
        pltpu.make_async_copy(vm_p.at[1], sh_den.at[vm_didx.at[0]],
                              sem_d1).wait()

        plsc.subcore_barrier()

        pltpu.sync_copy(sh_den.at[pl.ds(row0, ROWS_PER_TILE)], vm_den)

        def epi(t, carry):
            r0 = row0 + t * zc
            pltpu.sync_copy(sh_out.at[pl.ds(r0, zc)],
                            vm_rows.at[0, pl.ds(0, zc)])

            def erow(j, carry2):
                dv = plsc.load_gather(
                    vm_den, [jnp.full((16,), t * zc + j, jnp.int32)])
                rcp = 1.0 / (dv + EPS)
                for k in range(ch // 16):
                    val = (vm_rows[0, j, pl.ds(k * 16, 16)] * rcp
                           + vm_bias[pl.ds(k * 16, 16)])
                    if relu_out:
                        val = jnp.maximum(val, 0.0)
                    vm_rows[0, j, pl.ds(k * 16, 16)] = val
                return carry2
            lax.fori_loop(0, zc, erow, 0)
            pltpu.sync_copy(vm_rows.at[0, pl.ds(0, zc)],
                            outcat.at[pl.ds(hbase + r0, zc)])
            return carry
        lax.fori_loop(0, ROWS_PER_TILE // zc, epi, 0)

    return pl.kernel(
        body,
        out_type=jax.ShapeDtypeStruct((2 * NPAD, ch), jnp.float32),
        mesh=mesh,
        compiler_params=pltpu.CompilerParams(
            needs_layout_passes=False, use_tc_tiling_on_sc=False),
        scratch_types=[
            pltpu.VMEM((2 * cb, eb), jnp.int32),
            pltpu.VMEM((2 * cb, eb), jnp.int32),
            pltpu.VMEM((2, eb), jnp.float32),
            pltpu.VMEM((NPAD,), jnp.float32),
            pltpu.VMEM((NPAD,), jnp.float32),
            pltpu.VMEM((2, eb, ch), jnp.float32),
            pltpu.VMEM((ch,), jnp.float32),
            pltpu.VMEM((ROWS_PER_TILE,), jnp.float32),
            pltpu.VMEM_SHARED((NPAD, ch), jnp.float32),
            pltpu.VMEM_SHARED((NPAD,), jnp.float32),
            pltpu.SemaphoreType.DMA,
            pltpu.SemaphoreType.DMA,
            pltpu.SemaphoreType.DMA,
            pltpu.SemaphoreType.DMA,
            pltpu.SemaphoreType.DMA,
            pltpu.SemaphoreType.DMA,
        ],
    )


_sc_gat_l1 = _make_sc_gat(HID // 2, True, 64, 16)
_sc_gat_l2 = _make_sc_gat(NC // 2, False, 128, 8)


def kernel(x, edge_index, W1, a_src1, a_dst1, b1, W2, a_src2, a_dst2, b2):
    loops = jnp.arange(N_NODES, dtype=edge_index.dtype)
    ei = jnp.concatenate([edge_index, jnp.stack([loops, loops])], axis=1)
    fill = jnp.arange(EA_PAD - EA, dtype=jnp.int32) % N_NODES
    src = jnp.concatenate([ei[0], fill])
    dst = jnp.concatenate([ei[1], fill])
    src3a = src.reshape(NUM_TILES, ES // 64, 64)
    dst3a = dst.reshape(NUM_TILES, ES // 64, 64)
    src3b = src.reshape(NUM_TILES, ES // 128, 128)
    dst3b = dst.reshape(NUM_TILES, ES // 128, 128)

    x_pad = jnp.pad(x, ((0, NPAD - N_NODES), (0, 0)))

    av1 = jnp.stack([a_src1.reshape(HID), a_dst1.reshape(HID)])
    hl, hr, ab1 = _mm1(x_pad, W1, av1)
    hcat1 = jnp.concatenate([hl, hr], axis=0)
    in_cat = _sc_gat_l1(src3a, dst3a, ab1[0], ab1[1], hcat1,
                        b1.reshape(2, HID // 2))
    inl, inr = in_cat[:NPAD], in_cat[NPAD:]

    av2 = jnp.stack([a_src2.reshape(NC), a_dst2.reshape(NC)])
    h2, ab2 = _mm2(inl, inr, W2[: HID // 2], W2[HID // 2 :], av2)
    hcat2 = jnp.concatenate([h2[:, : NC // 2], h2[:, NC // 2 :]], axis=0)
    outcat = _sc_gat_l2(src3b, dst3b, ab2[0], ab2[1], hcat2,
                        b2.reshape(2, NC // 2))
    out = jnp.concatenate(
        [outcat[:N_NODES], outcat[NPAD : NPAD + N_NODES]], axis=1)
    return out
